# baseline (speedup 1.0000x reference)
import functools

import jax
import jax.numpy as jnp
from jax import lax
from jax.experimental import pallas as pl
from jax.experimental.pallas import tpu as pltpu
from jax.experimental.pallas import tpu_sc as plsc

_N = 10000
_E = 320000
_G = 64
_NC = 2
_NS = 16
_NW = _NC * _NS
_CH = 112
_N0 = 63
_N1 = 117
_MAXCH = max(_N0, _N1)
_EPAD = _NS * (_N0 + _N1) * _CH
_NBUF = 3
_SR = 624
_TAIL = _N - _SR * _NS


def _make_edge_agg(h):
    mesh = plsc.VectorSubcoreMesh(core_axis_name="c", subcore_axis_name="s")

    @functools.partial(
        pl.kernel,
        out_type=jax.ShapeDtypeStruct((_NC, _N, h), jnp.float32),
        mesh=mesh,
        scratch_types=[
            pltpu.VMEM((_NBUF, _CH), jnp.int32),
            pltpu.VMEM((_NBUF, _CH), jnp.int32),
            pltpu.VMEM((_NBUF, _CH), jnp.float32),
            pltpu.VMEM((_CH, h), jnp.float32),
            pltpu.VMEM((_CH, h), jnp.float32),
            pltpu.VMEM((_CH, h), jnp.float32),
            pltpu.VMEM_SHARED((_N, h), jnp.float32),
            pltpu.SemaphoreType.DMA,
            pltpu.SemaphoreType.DMA,
            pltpu.SemaphoreType.DMA,
            pltpu.SemaphoreType.DMA,
            pltpu.SemaphoreType.DMA,
            pltpu.SemaphoreType.DMA,
            pltpu.SemaphoreType.DMA,
            pltpu.SemaphoreType.DMA,
            pltpu.SemaphoreType.DMA,
            pltpu.SemaphoreType.DMA,
            pltpu.SemaphoreType.DMA,
            pltpu.SemaphoreType.DMA,
        ],
    )
    def edge_agg(y_hbm, src_hbm, dst_hbm, w_hbm, out_hbm,
                 srcb_v, dstb_v, wb_v, r0, r1, r2, acc_sh,
                 sg0, sg1, sg2, ss0, ss1, ss2, sc0, sc1, sc2,
                 sd0, sd1, sd2):
        rows = (r0, r1, r2)
        sg = (sg0, sg1, sg2)
        ss = (ss0, ss1, ss2)
        sc = (sc0, sc1, sc2)
        sd = (sd0, sd1, sd2)
        cid = lax.axis_index("c")
        sid = lax.axis_index("s")
        zero16 = jnp.zeros((16,), jnp.float32)

        nch = jnp.where(cid == 0, _N0, _N1)
        cbase = cid * _NS * _N0 + sid * nch

        def eoff(j):
            return pl.multiple_of((cbase + j) * _CH, 8)

        def srcload(j, b):
            return pltpu.make_async_copy(src_hbm.at[pl.ds(eoff(j), _CH)],
                                         srcb_v.at[b], sc[b])

        def dload(j, b):
            return pltpu.make_async_copy(dst_hbm.at[pl.ds(eoff(j), _CH)],
                                         dstb_v.at[b], sd[b])

        def wload(j, b):
            return pltpu.make_async_copy(w_hbm.at[pl.ds(eoff(j), _CH)],
                                         wb_v.at[b], sd[b])

        for b in range(_NBUF):
            srcload(b, b).start()
        for b in range(2):
            dload(b, b).start()
            wload(b, b).start()

        def zrow(rr, carry):
            for kk in range(h // 16):
                r0[rr, pl.ds(kk * 16, 16)] = zero16
            return carry
        lax.fori_loop(0, _CH, zrow, 0)

        row0 = sid * _SR
        nfull = _SR // _CH
        rem = _SR - nfull * _CH
        for i in range(nfull):
            pltpu.make_async_copy(
                r0, acc_sh.at[pl.ds(row0 + i * _CH, _CH)], ss0).start()
        pltpu.make_async_copy(
            r0.at[pl.ds(0, rem)],
            acc_sh.at[pl.ds(row0 + nfull * _CH, rem)], ss0).start()
        for i in range(nfull):
            pltpu.make_async_copy(
                r0, acc_sh.at[pl.ds(row0 + i * _CH, _CH)], ss0).wait()
        pltpu.make_async_copy(
            r0.at[pl.ds(0, rem)],
            acc_sh.at[pl.ds(row0 + nfull * _CH, rem)], ss0).wait()

        @pl.when(sid == 0)
        def _zero_tail():
            pltpu.sync_copy(r0.at[pl.ds(0, _TAIL)],
                            acc_sh.at[pl.ds(_SR * _NS, _TAIL)])
        plsc.subcore_barrier()

        def gat(j, b):
            return pltpu.make_async_copy(y_hbm.at[srcb_v.at[b]], rows[b],
                                         sg[b])

        def scat(b):
            return pltpu.make_async_copy(rows[b], acc_sh.at[dstb_v.at[b]],
                                         ss[b])

        def scale(b):
            rb = rows[b]

            def grp(g, c2):
                wv = wb_v[b, pl.ds(g * 16, 16)]
                for lane in range(16):
                    wgt = wv[lane]
                    e = g * 16 + lane
                    for kk in range(h // 16):
                        csl = pl.ds(kk * 16, 16)
                        rb[e, csl] = rb[e, csl] * wgt
                return c2
            lax.fori_loop(0, _CH // 16, grp, 0)

        srcload(0, 0).wait()
        gat(0, 0).start()
        srcload(1, 1).wait()
        gat(1, 1).start()

        def body(jj, carry):
            for b in range(_NBUF):
                j = jj * _NBUF + b

                @pl.when(j < nch)
                def _step():
                    gat(j, b).wait()
                    dload(j, b).wait()
                    wload(j, b).wait()
                    scale(b)
                    scat(b).start(add=True)
                    jf = j + 2
                    bf = (b + 2) % _NBUF

                    @pl.when(jf < nch)
                    def _issue():
                        @pl.when(j >= 1)
                        def _drain():
                            scat(bf).wait()
                        srcload(jf, bf).wait()
                        gat(jf, bf).start()
                        dload(jf, bf).start()
                        wload(jf, bf).start()

                    @pl.when(j + _NBUF < nch)
                    def _prefetch_src():
                        srcload(j + _NBUF, b).start()
            return carry
        lax.fori_loop(0, _MAXCH // _NBUF, body, 0)

        for b in range(_NBUF):
            scat(b).wait()
        plsc.subcore_barrier()

        for i in range(nfull + 1):
            cnt = _CH if i < nfull else rem
            b = i % 2
            row = row0 + i * _CH
            if i >= 2:
                pcnt = _CH if i - 2 < nfull else rem
                prow = row0 + (i - 2) * _CH
                pltpu.make_async_copy(
                    rows[b].at[pl.ds(0, pcnt)],
                    out_hbm.at[cid, pl.ds(prow, pcnt)], sg[b]).wait()
            pltpu.sync_copy(acc_sh.at[pl.ds(row, cnt)],
                            rows[b].at[pl.ds(0, cnt)])
            pltpu.make_async_copy(rows[b].at[pl.ds(0, cnt)],
                                  out_hbm.at[cid, pl.ds(row, cnt)],
                                  sg[b]).start()
        for i in (nfull - 1, nfull):
            cnt = _CH if i < nfull else rem
            b = i % 2
            row = row0 + i * _CH
            pltpu.make_async_copy(rows[b].at[pl.ds(0, cnt)],
                                  out_hbm.at[cid, pl.ds(row, cnt)],
                                  sg[b]).wait()

        @pl.when(sid == 0)
        def _read_tail():
            pltpu.sync_copy(acc_sh.at[pl.ds(_SR * _NS, _TAIL)],
                            r2.at[pl.ds(0, _TAIL)])
            pltpu.sync_copy(r2.at[pl.ds(0, _TAIL)],
                            out_hbm.at[cid, pl.ds(_SR * _NS, _TAIL)])

    return edge_agg


_edge_agg_cache = {}


def _edge_agg(h):
    if h not in _edge_agg_cache:
        _edge_agg_cache[h] = _make_edge_agg(h)
    return _edge_agg_cache[h]


def _tc_layer(acc, hprev, w_rel, w_root, b):
    hy = w_rel.shape[1]

    def body(a_ref, h_ref, wr_ref, wo_ref, b_ref, o_ref):
        agg = a_ref[0] + a_ref[1]
        o_ref[...] = jnp.tanh(
            jnp.dot(agg, wr_ref[...], preferred_element_type=jnp.float32)
            + b_ref[...]
            + jnp.dot(h_ref[...], wo_ref[...],
                      preferred_element_type=jnp.float32))

    return pl.pallas_call(
        body,
        out_shape=jax.ShapeDtypeStruct((_N, hy), jnp.float32),
    )(acc, hprev, w_rel, w_root, b.reshape(1, hy))


def _tc_final(acc, hprev, w_rel, w_root, b, batch, wm1, bm1, wm2, bm2,
              wm3, bm3):
    def body(a_ref, h_ref, wr_ref, wo_ref, b_ref, batch_ref,
             w1_ref, b1_ref, w2_ref, b2_ref, w3_ref, b3_ref, out_ref):
        agg = a_ref[0] + a_ref[1]
        hh = jnp.tanh(
            jnp.dot(agg, wr_ref[...], preferred_element_type=jnp.float32)
            + b_ref[...]
            + jnp.dot(h_ref[...], wo_ref[...],
                      preferred_element_type=jnp.float32))
        gids = lax.broadcasted_iota(jnp.int32, (_G, _N), 0)
        onehot = (batch_ref[...] == gids).astype(jnp.float32)
        pooled = jnp.dot(onehot, hh, preferred_element_type=jnp.float32)
        z1 = jnp.maximum(
            jnp.dot(pooled, w1_ref[...], preferred_element_type=jnp.float32)
            + b1_ref[...], 0.0)
        z2 = jnp.maximum(
            jnp.dot(z1, w2_ref[...], preferred_element_type=jnp.float32)
            + b2_ref[...], 0.0)
        out_ref[...] = (jnp.dot(z2, w3_ref[...], preferred_element_type=jnp.float32)
                        + b3_ref[...])

    return pl.pallas_call(
        body,
        out_shape=jax.ShapeDtypeStruct((_G, 1), jnp.float32),
    )(acc, hprev, w_rel, w_root, b.reshape(1, -1), batch.reshape(1, _N),
      wm1, bm1.reshape(1, -1), wm2, bm2.reshape(1, -1), wm3,
      bm3.reshape(1, 1))


def kernel(x, edge_index, batch, edge_attr,
           W1_rel, b1, W1_root, W2_rel, b2, W2_root, W3_rel, b3, W3_root,
           Wm1, bm1, Wm2, bm2, Wm3, bm3):
    pad = _EPAD - _E
    src = jnp.concatenate([edge_index[0], jnp.zeros((pad,), jnp.int32)])
    dst = jnp.concatenate([edge_index[1], jnp.zeros((pad,), jnp.int32)])
    w = jnp.concatenate([edge_attr, jnp.zeros((pad,), jnp.float32)])

    agg = _edge_agg(128)
    acc1 = agg(x, src, dst, w)
    h1 = _tc_layer(acc1, x, W1_rel, W1_root, b1)
    acc2 = agg(h1, src, dst, w)
    h2 = _tc_layer(acc2, h1, W2_rel, W2_root, b2)
    acc3 = agg(h2, src, dst, w)
    return _tc_final(acc3, h2, W3_rel, W3_root, b3, batch,
                     Wm1, bm1, Wm2, bm2, Wm3, bm3)


# speedup vs baseline: 1.1899x; 1.1899x over previous
import functools

import jax
import jax.numpy as jnp
from jax import lax
from jax.experimental import pallas as pl
from jax.experimental.pallas import tpu as pltpu
from jax.experimental.pallas import tpu_sc as plsc

_N = 10000
_E = 320000
_G = 64
_NC = 2
_NS = 16
_NW = _NC * _NS
_CH = 112
_N0 = 117
_N1 = 63
_MAXCH = max(_N0, _N1)
_EPAD = _NS * (_N0 + _N1) * _CH
_NBUF = 3
_SR = 624
_TAIL = _N - _SR * _NS


def _make_edge_agg(h):
    mesh = plsc.VectorSubcoreMesh(core_axis_name="c", subcore_axis_name="s")

    @functools.partial(
        pl.kernel,
        out_type=jax.ShapeDtypeStruct((_NC, _N, h), jnp.float32),
        mesh=mesh,
        scratch_types=[
            pltpu.VMEM((_NBUF, _CH), jnp.int32),
            pltpu.VMEM((_NBUF, _CH), jnp.int32),
            pltpu.VMEM((_NBUF, _CH), jnp.float32),
            pltpu.VMEM((_CH, h), jnp.float32),
            pltpu.VMEM((_CH, h), jnp.float32),
            pltpu.VMEM((_CH, h), jnp.float32),
            pltpu.VMEM_SHARED((_N, h), jnp.float32),
            pltpu.SemaphoreType.DMA,
            pltpu.SemaphoreType.DMA,
            pltpu.SemaphoreType.DMA,
            pltpu.SemaphoreType.DMA,
            pltpu.SemaphoreType.DMA,
            pltpu.SemaphoreType.DMA,
            pltpu.SemaphoreType.DMA,
            pltpu.SemaphoreType.DMA,
            pltpu.SemaphoreType.DMA,
            pltpu.SemaphoreType.DMA,
            pltpu.SemaphoreType.DMA,
            pltpu.SemaphoreType.DMA,
        ],
    )
    def edge_agg(y_hbm, src_hbm, dst_hbm, w_hbm, out_hbm,
                 srcb_v, dstb_v, wb_v, r0, r1, r2, acc_sh,
                 sg0, sg1, sg2, ss0, ss1, ss2, sc0, sc1, sc2,
                 sd0, sd1, sd2):
        rows = (r0, r1, r2)
        sg = (sg0, sg1, sg2)
        ss = (ss0, ss1, ss2)
        sc = (sc0, sc1, sc2)
        sd = (sd0, sd1, sd2)
        cid = lax.axis_index("c")
        sid = lax.axis_index("s")
        zero16 = jnp.zeros((16,), jnp.float32)

        nch = jnp.where(cid == 0, _N0, _N1)
        cbase = cid * _NS * _N0 + sid * nch

        def eoff(j):
            return pl.multiple_of((cbase + j) * _CH, 8)

        def srcload(j, b):
            return pltpu.make_async_copy(src_hbm.at[pl.ds(eoff(j), _CH)],
                                         srcb_v.at[b], sc[b])

        def dload(j, b):
            return pltpu.make_async_copy(dst_hbm.at[pl.ds(eoff(j), _CH)],
                                         dstb_v.at[b], sd[b])

        def wload(j, b):
            return pltpu.make_async_copy(w_hbm.at[pl.ds(eoff(j), _CH)],
                                         wb_v.at[b], sd[b])

        for b in range(_NBUF):
            srcload(b, b).start()
        for b in range(2):
            dload(b, b).start()
            wload(b, b).start()

        def zrow(rr, carry):
            for kk in range(h // 16):
                r0[rr, pl.ds(kk * 16, 16)] = zero16
            return carry
        lax.fori_loop(0, _CH, zrow, 0)

        row0 = sid * _SR
        nfull = _SR // _CH
        rem = _SR - nfull * _CH
        for i in range(nfull):
            pltpu.make_async_copy(
                r0, acc_sh.at[pl.ds(row0 + i * _CH, _CH)], ss0).start()
        pltpu.make_async_copy(
            r0.at[pl.ds(0, rem)],
            acc_sh.at[pl.ds(row0 + nfull * _CH, rem)], ss0).start()
        for i in range(nfull):
            pltpu.make_async_copy(
                r0, acc_sh.at[pl.ds(row0 + i * _CH, _CH)], ss0).wait()
        pltpu.make_async_copy(
            r0.at[pl.ds(0, rem)],
            acc_sh.at[pl.ds(row0 + nfull * _CH, rem)], ss0).wait()

        @pl.when(sid == 0)
        def _zero_tail():
            pltpu.sync_copy(r0.at[pl.ds(0, _TAIL)],
                            acc_sh.at[pl.ds(_SR * _NS, _TAIL)])
        plsc.subcore_barrier()

        def gat(j, b):
            return pltpu.make_async_copy(y_hbm.at[srcb_v.at[b]], rows[b],
                                         sg[b])

        def scat(b):
            return pltpu.make_async_copy(rows[b], acc_sh.at[dstb_v.at[b]],
                                         ss[b])

        def scale(b):
            rb = rows[b]

            def grp(g, c2):
                wv = wb_v[b, pl.ds(g * 16, 16)]
                for lane in range(16):
                    wgt = wv[lane]
                    e = g * 16 + lane
                    for kk in range(h // 16):
                        csl = pl.ds(kk * 16, 16)
                        rb[e, csl] = rb[e, csl] * wgt
                return c2
            lax.fori_loop(0, _CH // 16, grp, 0)

        srcload(0, 0).wait()
        gat(0, 0).start()
        srcload(1, 1).wait()
        gat(1, 1).start()

        def body(jj, carry):
            for b in range(_NBUF):
                j = jj * _NBUF + b

                @pl.when(j < nch)
                def _step():
                    gat(j, b).wait()
                    dload(j, b).wait()
                    wload(j, b).wait()
                    scale(b)
                    scat(b).start(add=True)
                    jf = j + 2
                    bf = (b + 2) % _NBUF

                    @pl.when(jf < nch)
                    def _issue():
                        @pl.when(j >= 1)
                        def _drain():
                            scat(bf).wait()
                        srcload(jf, bf).wait()
                        gat(jf, bf).start()
                        dload(jf, bf).start()
                        wload(jf, bf).start()

                    @pl.when(j + _NBUF < nch)
                    def _prefetch_src():
                        srcload(j + _NBUF, b).start()
            return carry
        lax.fori_loop(0, _MAXCH // _NBUF, body, 0)

        for b in range(_NBUF):
            scat(b).wait()
        plsc.subcore_barrier()

        for i in range(nfull + 1):
            cnt = _CH if i < nfull else rem
            b = i % 2
            row = row0 + i * _CH
            if i >= 2:
                pcnt = _CH if i - 2 < nfull else rem
                prow = row0 + (i - 2) * _CH
                pltpu.make_async_copy(
                    rows[b].at[pl.ds(0, pcnt)],
                    out_hbm.at[cid, pl.ds(prow, pcnt)], sg[b]).wait()
            pltpu.sync_copy(acc_sh.at[pl.ds(row, cnt)],
                            rows[b].at[pl.ds(0, cnt)])
            pltpu.make_async_copy(rows[b].at[pl.ds(0, cnt)],
                                  out_hbm.at[cid, pl.ds(row, cnt)],
                                  sg[b]).start()
        for i in (nfull - 1, nfull):
            cnt = _CH if i < nfull else rem
            b = i % 2
            row = row0 + i * _CH
            pltpu.make_async_copy(rows[b].at[pl.ds(0, cnt)],
                                  out_hbm.at[cid, pl.ds(row, cnt)],
                                  sg[b]).wait()

        @pl.when(sid == 0)
        def _read_tail():
            pltpu.sync_copy(acc_sh.at[pl.ds(_SR * _NS, _TAIL)],
                            r2.at[pl.ds(0, _TAIL)])
            pltpu.sync_copy(r2.at[pl.ds(0, _TAIL)],
                            out_hbm.at[cid, pl.ds(_SR * _NS, _TAIL)])

    return edge_agg


_edge_agg_cache = {}


def _edge_agg(h):
    if h not in _edge_agg_cache:
        _edge_agg_cache[h] = _make_edge_agg(h)
    return _edge_agg_cache[h]


def _tc_layer(acc, hprev, w_rel, w_root, b):
    hy = w_rel.shape[1]

    def body(a_ref, h_ref, wr_ref, wo_ref, b_ref, o_ref):
        agg = a_ref[0] + a_ref[1]
        o_ref[...] = jnp.tanh(
            jnp.dot(agg, wr_ref[...], preferred_element_type=jnp.float32)
            + b_ref[...]
            + jnp.dot(h_ref[...], wo_ref[...],
                      preferred_element_type=jnp.float32))

    return pl.pallas_call(
        body,
        out_shape=jax.ShapeDtypeStruct((_N, hy), jnp.float32),
    )(acc, hprev, w_rel, w_root, b.reshape(1, hy))


def _tc_final(acc, hprev, w_rel, w_root, b, batch, wm1, bm1, wm2, bm2,
              wm3, bm3):
    def body(a_ref, h_ref, wr_ref, wo_ref, b_ref, batch_ref,
             w1_ref, b1_ref, w2_ref, b2_ref, w3_ref, b3_ref, out_ref):
        agg = a_ref[0] + a_ref[1]
        hh = jnp.tanh(
            jnp.dot(agg, wr_ref[...], preferred_element_type=jnp.float32)
            + b_ref[...]
            + jnp.dot(h_ref[...], wo_ref[...],
                      preferred_element_type=jnp.float32))
        gids = lax.broadcasted_iota(jnp.int32, (_G, _N), 0)
        onehot = (batch_ref[...] == gids).astype(jnp.float32)
        pooled = jnp.dot(onehot, hh, preferred_element_type=jnp.float32)
        z1 = jnp.maximum(
            jnp.dot(pooled, w1_ref[...], preferred_element_type=jnp.float32)
            + b1_ref[...], 0.0)
        z2 = jnp.maximum(
            jnp.dot(z1, w2_ref[...], preferred_element_type=jnp.float32)
            + b2_ref[...], 0.0)
        out_ref[...] = (jnp.dot(z2, w3_ref[...], preferred_element_type=jnp.float32)
                        + b3_ref[...])

    return pl.pallas_call(
        body,
        out_shape=jax.ShapeDtypeStruct((_G, 1), jnp.float32),
    )(acc, hprev, w_rel, w_root, b.reshape(1, -1), batch.reshape(1, _N),
      wm1, bm1.reshape(1, -1), wm2, bm2.reshape(1, -1), wm3,
      bm3.reshape(1, 1))


def kernel(x, edge_index, batch, edge_attr,
           W1_rel, b1, W1_root, W2_rel, b2, W2_root, W3_rel, b3, W3_root,
           Wm1, bm1, Wm2, bm2, Wm3, bm3):
    pad = _EPAD - _E
    src = jnp.concatenate([edge_index[0], jnp.zeros((pad,), jnp.int32)])
    dst = jnp.concatenate([edge_index[1], jnp.zeros((pad,), jnp.int32)])
    w = jnp.concatenate([edge_attr, jnp.zeros((pad,), jnp.float32)])

    agg = _edge_agg(128)
    acc1 = agg(x, src, dst, w)
    h1 = _tc_layer(acc1, x, W1_rel, W1_root, b1)
    acc2 = agg(h1, src, dst, w)
    h2 = _tc_layer(acc2, h1, W2_rel, W2_root, b2)
    acc3 = agg(h2, src, dst, w)
    return _tc_final(acc3, h2, W3_rel, W3_root, b3, batch,
                     Wm1, bm1, Wm2, bm2, Wm3, bm3)


# speedup vs baseline: 1.2370x; 1.0396x over previous
import functools

import jax
import jax.numpy as jnp
from jax import lax
from jax.experimental import pallas as pl
from jax.experimental.pallas import tpu as pltpu
from jax.experimental.pallas import tpu_sc as plsc

_N = 10000
_E = 320000
_G = 64
_NC = 2
_NS = 16
_NW = _NC * _NS
_CH = 112
_N0 = 129
_N1 = 51
_MAXCH = max(_N0, _N1)
_EPAD = _NS * (_N0 + _N1) * _CH
_NBUF = 3
_SR = 624
_TAIL = _N - _SR * _NS


def _make_edge_agg(h):
    mesh = plsc.VectorSubcoreMesh(core_axis_name="c", subcore_axis_name="s")

    @functools.partial(
        pl.kernel,
        out_type=jax.ShapeDtypeStruct((_NC, _N, h), jnp.float32),
        mesh=mesh,
        scratch_types=[
            pltpu.VMEM((_NBUF, _CH), jnp.int32),
            pltpu.VMEM((_NBUF, _CH), jnp.int32),
            pltpu.VMEM((_NBUF, _CH), jnp.float32),
            pltpu.VMEM((_CH, h), jnp.float32),
            pltpu.VMEM((_CH, h), jnp.float32),
            pltpu.VMEM((_CH, h), jnp.float32),
            pltpu.VMEM_SHARED((_N, h), jnp.float32),
            pltpu.SemaphoreType.DMA,
            pltpu.SemaphoreType.DMA,
            pltpu.SemaphoreType.DMA,
            pltpu.SemaphoreType.DMA,
            pltpu.SemaphoreType.DMA,
            pltpu.SemaphoreType.DMA,
            pltpu.SemaphoreType.DMA,
            pltpu.SemaphoreType.DMA,
            pltpu.SemaphoreType.DMA,
            pltpu.SemaphoreType.DMA,
            pltpu.SemaphoreType.DMA,
            pltpu.SemaphoreType.DMA,
        ],
    )
    def edge_agg(y_hbm, src_hbm, dst_hbm, w_hbm, out_hbm,
                 srcb_v, dstb_v, wb_v, r0, r1, r2, acc_sh,
                 sg0, sg1, sg2, ss0, ss1, ss2, sc0, sc1, sc2,
                 sd0, sd1, sd2):
        rows = (r0, r1, r2)
        sg = (sg0, sg1, sg2)
        ss = (ss0, ss1, ss2)
        sc = (sc0, sc1, sc2)
        sd = (sd0, sd1, sd2)
        cid = lax.axis_index("c")
        sid = lax.axis_index("s")
        zero16 = jnp.zeros((16,), jnp.float32)

        nch = jnp.where(cid == 0, _N0, _N1)
        cbase = cid * _NS * _N0 + sid * nch

        def eoff(j):
            return pl.multiple_of((cbase + j) * _CH, 8)

        def srcload(j, b):
            return pltpu.make_async_copy(src_hbm.at[pl.ds(eoff(j), _CH)],
                                         srcb_v.at[b], sc[b])

        def dload(j, b):
            return pltpu.make_async_copy(dst_hbm.at[pl.ds(eoff(j), _CH)],
                                         dstb_v.at[b], sd[b])

        def wload(j, b):
            return pltpu.make_async_copy(w_hbm.at[pl.ds(eoff(j), _CH)],
                                         wb_v.at[b], sd[b])

        for b in range(_NBUF):
            srcload(b, b).start()
        for b in range(2):
            dload(b, b).start()
            wload(b, b).start()

        def zrow(rr, carry):
            for kk in range(h // 16):
                r0[rr, pl.ds(kk * 16, 16)] = zero16
            return carry
        lax.fori_loop(0, _CH, zrow, 0)

        row0 = sid * _SR
        nfull = _SR // _CH
        rem = _SR - nfull * _CH
        for i in range(nfull):
            pltpu.make_async_copy(
                r0, acc_sh.at[pl.ds(row0 + i * _CH, _CH)], ss0).start()
        pltpu.make_async_copy(
            r0.at[pl.ds(0, rem)],
            acc_sh.at[pl.ds(row0 + nfull * _CH, rem)], ss0).start()
        for i in range(nfull):
            pltpu.make_async_copy(
                r0, acc_sh.at[pl.ds(row0 + i * _CH, _CH)], ss0).wait()
        pltpu.make_async_copy(
            r0.at[pl.ds(0, rem)],
            acc_sh.at[pl.ds(row0 + nfull * _CH, rem)], ss0).wait()

        @pl.when(sid == 0)
        def _zero_tail():
            pltpu.sync_copy(r0.at[pl.ds(0, _TAIL)],
                            acc_sh.at[pl.ds(_SR * _NS, _TAIL)])
        plsc.subcore_barrier()

        def gat(j, b):
            return pltpu.make_async_copy(y_hbm.at[srcb_v.at[b]], rows[b],
                                         sg[b])

        def scat(b):
            return pltpu.make_async_copy(rows[b], acc_sh.at[dstb_v.at[b]],
                                         ss[b])

        def scale(b):
            rb = rows[b]

            def grp(g, c2):
                wv = wb_v[b, pl.ds(g * 16, 16)]
                for lane in range(16):
                    wgt = wv[lane]
                    e = g * 16 + lane
                    for kk in range(h // 16):
                        csl = pl.ds(kk * 16, 16)
                        rb[e, csl] = rb[e, csl] * wgt
                return c2
            lax.fori_loop(0, _CH // 16, grp, 0)

        srcload(0, 0).wait()
        gat(0, 0).start()
        srcload(1, 1).wait()
        gat(1, 1).start()

        def body(jj, carry):
            for b in range(_NBUF):
                j = jj * _NBUF + b

                @pl.when(j < nch)
                def _step():
                    gat(j, b).wait()
                    dload(j, b).wait()
                    wload(j, b).wait()
                    scale(b)
                    scat(b).start(add=True)
                    jf = j + 2
                    bf = (b + 2) % _NBUF

                    @pl.when(jf < nch)
                    def _issue():
                        @pl.when(j >= 1)
                        def _drain():
                            scat(bf).wait()
                        srcload(jf, bf).wait()
                        gat(jf, bf).start()
                        dload(jf, bf).start()
                        wload(jf, bf).start()

                    @pl.when(j + _NBUF < nch)
                    def _prefetch_src():
                        srcload(j + _NBUF, b).start()
            return carry
        lax.fori_loop(0, _MAXCH // _NBUF, body, 0)

        for b in range(_NBUF):
            scat(b).wait()
        plsc.subcore_barrier()

        for i in range(nfull + 1):
            cnt = _CH if i < nfull else rem
            b = i % 2
            row = row0 + i * _CH
            if i >= 2:
                pcnt = _CH if i - 2 < nfull else rem
                prow = row0 + (i - 2) * _CH
                pltpu.make_async_copy(
                    rows[b].at[pl.ds(0, pcnt)],
                    out_hbm.at[cid, pl.ds(prow, pcnt)], sg[b]).wait()
            pltpu.sync_copy(acc_sh.at[pl.ds(row, cnt)],
                            rows[b].at[pl.ds(0, cnt)])
            pltpu.make_async_copy(rows[b].at[pl.ds(0, cnt)],
                                  out_hbm.at[cid, pl.ds(row, cnt)],
                                  sg[b]).start()
        for i in (nfull - 1, nfull):
            cnt = _CH if i < nfull else rem
            b = i % 2
            row = row0 + i * _CH
            pltpu.make_async_copy(rows[b].at[pl.ds(0, cnt)],
                                  out_hbm.at[cid, pl.ds(row, cnt)],
                                  sg[b]).wait()

        @pl.when(sid == 0)
        def _read_tail():
            pltpu.sync_copy(acc_sh.at[pl.ds(_SR * _NS, _TAIL)],
                            r2.at[pl.ds(0, _TAIL)])
            pltpu.sync_copy(r2.at[pl.ds(0, _TAIL)],
                            out_hbm.at[cid, pl.ds(_SR * _NS, _TAIL)])

    return edge_agg


_edge_agg_cache = {}


def _edge_agg(h):
    if h not in _edge_agg_cache:
        _edge_agg_cache[h] = _make_edge_agg(h)
    return _edge_agg_cache[h]


def _tc_layer(acc, hprev, w_rel, w_root, b):
    hy = w_rel.shape[1]

    def body(a_ref, h_ref, wr_ref, wo_ref, b_ref, o_ref):
        agg = a_ref[0] + a_ref[1]
        o_ref[...] = jnp.tanh(
            jnp.dot(agg, wr_ref[...], preferred_element_type=jnp.float32)
            + b_ref[...]
            + jnp.dot(h_ref[...], wo_ref[...],
                      preferred_element_type=jnp.float32))

    return pl.pallas_call(
        body,
        out_shape=jax.ShapeDtypeStruct((_N, hy), jnp.float32),
    )(acc, hprev, w_rel, w_root, b.reshape(1, hy))


def _tc_final(acc, hprev, w_rel, w_root, b, batch, wm1, bm1, wm2, bm2,
              wm3, bm3):
    def body(a_ref, h_ref, wr_ref, wo_ref, b_ref, batch_ref,
             w1_ref, b1_ref, w2_ref, b2_ref, w3_ref, b3_ref, out_ref):
        agg = a_ref[0] + a_ref[1]
        hh = jnp.tanh(
            jnp.dot(agg, wr_ref[...], preferred_element_type=jnp.float32)
            + b_ref[...]
            + jnp.dot(h_ref[...], wo_ref[...],
                      preferred_element_type=jnp.float32))
        gids = lax.broadcasted_iota(jnp.int32, (_G, _N), 0)
        onehot = (batch_ref[...] == gids).astype(jnp.float32)
        pooled = jnp.dot(onehot, hh, preferred_element_type=jnp.float32)
        z1 = jnp.maximum(
            jnp.dot(pooled, w1_ref[...], preferred_element_type=jnp.float32)
            + b1_ref[...], 0.0)
        z2 = jnp.maximum(
            jnp.dot(z1, w2_ref[...], preferred_element_type=jnp.float32)
            + b2_ref[...], 0.0)
        out_ref[...] = (jnp.dot(z2, w3_ref[...], preferred_element_type=jnp.float32)
                        + b3_ref[...])

    return pl.pallas_call(
        body,
        out_shape=jax.ShapeDtypeStruct((_G, 1), jnp.float32),
    )(acc, hprev, w_rel, w_root, b.reshape(1, -1), batch.reshape(1, _N),
      wm1, bm1.reshape(1, -1), wm2, bm2.reshape(1, -1), wm3,
      bm3.reshape(1, 1))


def kernel(x, edge_index, batch, edge_attr,
           W1_rel, b1, W1_root, W2_rel, b2, W2_root, W3_rel, b3, W3_root,
           Wm1, bm1, Wm2, bm2, Wm3, bm3):
    pad = _EPAD - _E
    src = jnp.concatenate([edge_index[0], jnp.zeros((pad,), jnp.int32)])
    dst = jnp.concatenate([edge_index[1], jnp.zeros((pad,), jnp.int32)])
    w = jnp.concatenate([edge_attr, jnp.zeros((pad,), jnp.float32)])

    agg = _edge_agg(128)
    acc1 = agg(x, src, dst, w)
    h1 = _tc_layer(acc1, x, W1_rel, W1_root, b1)
    acc2 = agg(h1, src, dst, w)
    h2 = _tc_layer(acc2, h1, W2_rel, W2_root, b2)
    acc3 = agg(h2, src, dst, w)
    return _tc_final(acc3, h2, W3_rel, W3_root, b3, batch,
                     Wm1, bm1, Wm2, bm2, Wm3, bm3)


# speedup vs baseline: 1.2371x; 1.0000x over previous
import functools

import jax
import jax.numpy as jnp
from jax import lax
from jax.experimental import pallas as pl
from jax.experimental.pallas import tpu as pltpu
from jax.experimental.pallas import tpu_sc as plsc

_N = 10000
_E = 320000
_G = 64
_NC = 2
_NS = 16
_NW = _NC * _NS
_CH = 64
_N0 = 225
_N1 = 90
_MAXCH = max(_N0, _N1)
_EPAD = _NS * (_N0 + _N1) * _CH
_NBUF = 5
_DIST = 4
_SR = 624
_TAIL = _N - _SR * _NS


def _make_edge_agg(h):
    mesh = plsc.VectorSubcoreMesh(core_axis_name="c", subcore_axis_name="s")

    @functools.partial(
        pl.kernel,
        out_type=jax.ShapeDtypeStruct((_NC, _N, h), jnp.float32),
        mesh=mesh,
        scratch_types=[
            pltpu.VMEM((_NBUF, _CH), jnp.int32),
            pltpu.VMEM((_NBUF, _CH), jnp.int32),
            pltpu.VMEM((_NBUF, _CH), jnp.float32),
            pltpu.VMEM((_CH, h), jnp.float32),
            pltpu.VMEM((_CH, h), jnp.float32),
            pltpu.VMEM((_CH, h), jnp.float32),
            pltpu.VMEM((_CH, h), jnp.float32),
            pltpu.VMEM((_CH, h), jnp.float32),
            pltpu.VMEM_SHARED((_N, h), jnp.float32),
            pltpu.SemaphoreType.DMA,
            pltpu.SemaphoreType.DMA,
            pltpu.SemaphoreType.DMA,
            pltpu.SemaphoreType.DMA,
            pltpu.SemaphoreType.DMA,
            pltpu.SemaphoreType.DMA,
            pltpu.SemaphoreType.DMA,
            pltpu.SemaphoreType.DMA,
            pltpu.SemaphoreType.DMA,
            pltpu.SemaphoreType.DMA,
            pltpu.SemaphoreType.DMA,
            pltpu.SemaphoreType.DMA,
            pltpu.SemaphoreType.DMA,
            pltpu.SemaphoreType.DMA,
            pltpu.SemaphoreType.DMA,
            pltpu.SemaphoreType.DMA,
            pltpu.SemaphoreType.DMA,
            pltpu.SemaphoreType.DMA,
            pltpu.SemaphoreType.DMA,
            pltpu.SemaphoreType.DMA,
        ],
    )
    def edge_agg(y_hbm, src_hbm, dst_hbm, w_hbm, out_hbm,
                 srcb_v, dstb_v, wb_v, r0, r1, r2, r3, r4, acc_sh,
                 sg0, sg1, sg2, sg3, sg4, ss0, ss1, ss2, ss3, ss4,
                 sc0, sc1, sc2, sc3, sc4, sd0, sd1, sd2, sd3, sd4):
        rows = (r0, r1, r2, r3, r4)
        sg = (sg0, sg1, sg2, sg3, sg4)
        ss = (ss0, ss1, ss2, ss3, ss4)
        sc = (sc0, sc1, sc2, sc3, sc4)
        sd = (sd0, sd1, sd2, sd3, sd4)
        cid = lax.axis_index("c")
        sid = lax.axis_index("s")
        zero16 = jnp.zeros((16,), jnp.float32)

        nch = jnp.where(cid == 0, _N0, _N1)
        cbase = cid * _NS * _N0 + sid * nch

        def eoff(j):
            return pl.multiple_of((cbase + j) * _CH, 8)

        def srcload(j, b):
            return pltpu.make_async_copy(src_hbm.at[pl.ds(eoff(j), _CH)],
                                         srcb_v.at[b], sc[b])

        def dload(j, b):
            return pltpu.make_async_copy(dst_hbm.at[pl.ds(eoff(j), _CH)],
                                         dstb_v.at[b], sd[b])

        def wload(j, b):
            return pltpu.make_async_copy(w_hbm.at[pl.ds(eoff(j), _CH)],
                                         wb_v.at[b], sd[b])

        for b in range(_NBUF):
            srcload(b, b).start()
        for b in range(_DIST):
            dload(b, b).start()
            wload(b, b).start()

        def zrow(rr, carry):
            for kk in range(h // 16):
                r0[rr, pl.ds(kk * 16, 16)] = zero16
            return carry
        lax.fori_loop(0, _CH, zrow, 0)

        row0 = sid * _SR
        nfull = _SR // _CH
        rem = _SR - nfull * _CH
        for i in range(nfull):
            pltpu.make_async_copy(
                r0, acc_sh.at[pl.ds(row0 + i * _CH, _CH)], ss0).start()
        pltpu.make_async_copy(
            r0.at[pl.ds(0, rem)],
            acc_sh.at[pl.ds(row0 + nfull * _CH, rem)], ss0).start()
        for i in range(nfull):
            pltpu.make_async_copy(
                r0, acc_sh.at[pl.ds(row0 + i * _CH, _CH)], ss0).wait()
        pltpu.make_async_copy(
            r0.at[pl.ds(0, rem)],
            acc_sh.at[pl.ds(row0 + nfull * _CH, rem)], ss0).wait()

        @pl.when(sid == 0)
        def _zero_tail():
            pltpu.sync_copy(r0.at[pl.ds(0, _TAIL)],
                            acc_sh.at[pl.ds(_SR * _NS, _TAIL)])
        plsc.subcore_barrier()

        def gat(j, b):
            return pltpu.make_async_copy(y_hbm.at[srcb_v.at[b]], rows[b],
                                         sg[b])

        def scat(b):
            return pltpu.make_async_copy(rows[b], acc_sh.at[dstb_v.at[b]],
                                         ss[b])

        def scale(b):
            rb = rows[b]

            def grp(g, c2):
                wv = wb_v[b, pl.ds(g * 16, 16)]
                for lane in range(16):
                    wgt = wv[lane]
                    e = g * 16 + lane
                    for kk in range(h // 16):
                        csl = pl.ds(kk * 16, 16)
                        rb[e, csl] = rb[e, csl] * wgt
                return c2
            lax.fori_loop(0, _CH // 16, grp, 0)

        for b in range(_DIST):
            srcload(b, b).wait()
            gat(b, b).start()

        def body(jj, carry):
            for b in range(_NBUF):
                j = jj * _NBUF + b

                @pl.when(j < nch)
                def _step():
                    gat(j, b).wait()
                    dload(j, b).wait()
                    wload(j, b).wait()
                    scale(b)
                    scat(b).start(add=True)
                    jf = j + _DIST
                    bf = (b + _DIST) % _NBUF

                    @pl.when(jf < nch)
                    def _issue():
                        @pl.when(j >= 1)
                        def _drain():
                            scat(bf).wait()
                        srcload(jf, bf).wait()
                        gat(jf, bf).start()
                        dload(jf, bf).start()
                        wload(jf, bf).start()

                    @pl.when(j + _NBUF < nch)
                    def _prefetch_src():
                        srcload(j + _NBUF, b).start()
            return carry
        lax.fori_loop(0, _MAXCH // _NBUF, body, 0)

        for b in range(_NBUF):
            scat(b).wait()
        plsc.subcore_barrier()

        for i in range(nfull + 1):
            cnt = _CH if i < nfull else rem
            b = i % 2
            row = row0 + i * _CH
            if i >= 2:
                pcnt = _CH if i - 2 < nfull else rem
                prow = row0 + (i - 2) * _CH
                pltpu.make_async_copy(
                    rows[b].at[pl.ds(0, pcnt)],
                    out_hbm.at[cid, pl.ds(prow, pcnt)], sg[b]).wait()
            pltpu.sync_copy(acc_sh.at[pl.ds(row, cnt)],
                            rows[b].at[pl.ds(0, cnt)])
            pltpu.make_async_copy(rows[b].at[pl.ds(0, cnt)],
                                  out_hbm.at[cid, pl.ds(row, cnt)],
                                  sg[b]).start()
        for i in (nfull - 1, nfull):
            cnt = _CH if i < nfull else rem
            b = i % 2
            row = row0 + i * _CH
            pltpu.make_async_copy(rows[b].at[pl.ds(0, cnt)],
                                  out_hbm.at[cid, pl.ds(row, cnt)],
                                  sg[b]).wait()

        @pl.when(sid == 0)
        def _read_tail():
            pltpu.sync_copy(acc_sh.at[pl.ds(_SR * _NS, _TAIL)],
                            r2.at[pl.ds(0, _TAIL)])
            pltpu.sync_copy(r2.at[pl.ds(0, _TAIL)],
                            out_hbm.at[cid, pl.ds(_SR * _NS, _TAIL)])

    return edge_agg


_edge_agg_cache = {}


def _edge_agg(h):
    if h not in _edge_agg_cache:
        _edge_agg_cache[h] = _make_edge_agg(h)
    return _edge_agg_cache[h]


def _tc_layer(acc, hprev, w_rel, w_root, b):
    hy = w_rel.shape[1]

    def body(a_ref, h_ref, wr_ref, wo_ref, b_ref, o_ref):
        agg = a_ref[0] + a_ref[1]
        o_ref[...] = jnp.tanh(
            jnp.dot(agg, wr_ref[...], preferred_element_type=jnp.float32)
            + b_ref[...]
            + jnp.dot(h_ref[...], wo_ref[...],
                      preferred_element_type=jnp.float32))

    return pl.pallas_call(
        body,
        out_shape=jax.ShapeDtypeStruct((_N, hy), jnp.float32),
    )(acc, hprev, w_rel, w_root, b.reshape(1, hy))


def _tc_final(acc, hprev, w_rel, w_root, b, batch, wm1, bm1, wm2, bm2,
              wm3, bm3):
    def body(a_ref, h_ref, wr_ref, wo_ref, b_ref, batch_ref,
             w1_ref, b1_ref, w2_ref, b2_ref, w3_ref, b3_ref, out_ref):
        agg = a_ref[0] + a_ref[1]
        hh = jnp.tanh(
            jnp.dot(agg, wr_ref[...], preferred_element_type=jnp.float32)
            + b_ref[...]
            + jnp.dot(h_ref[...], wo_ref[...],
                      preferred_element_type=jnp.float32))
        gids = lax.broadcasted_iota(jnp.int32, (_G, _N), 0)
        onehot = (batch_ref[...] == gids).astype(jnp.float32)
        pooled = jnp.dot(onehot, hh, preferred_element_type=jnp.float32)
        z1 = jnp.maximum(
            jnp.dot(pooled, w1_ref[...], preferred_element_type=jnp.float32)
            + b1_ref[...], 0.0)
        z2 = jnp.maximum(
            jnp.dot(z1, w2_ref[...], preferred_element_type=jnp.float32)
            + b2_ref[...], 0.0)
        out_ref[...] = (jnp.dot(z2, w3_ref[...], preferred_element_type=jnp.float32)
                        + b3_ref[...])

    return pl.pallas_call(
        body,
        out_shape=jax.ShapeDtypeStruct((_G, 1), jnp.float32),
    )(acc, hprev, w_rel, w_root, b.reshape(1, -1), batch.reshape(1, _N),
      wm1, bm1.reshape(1, -1), wm2, bm2.reshape(1, -1), wm3,
      bm3.reshape(1, 1))


def kernel(x, edge_index, batch, edge_attr,
           W1_rel, b1, W1_root, W2_rel, b2, W2_root, W3_rel, b3, W3_root,
           Wm1, bm1, Wm2, bm2, Wm3, bm3):
    pad = _EPAD - _E
    src = jnp.concatenate([edge_index[0], jnp.zeros((pad,), jnp.int32)])
    dst = jnp.concatenate([edge_index[1], jnp.zeros((pad,), jnp.int32)])
    w = jnp.concatenate([edge_attr, jnp.zeros((pad,), jnp.float32)])

    agg = _edge_agg(128)
    acc1 = agg(x, src, dst, w)
    h1 = _tc_layer(acc1, x, W1_rel, W1_root, b1)
    acc2 = agg(h1, src, dst, w)
    h2 = _tc_layer(acc2, h1, W2_rel, W2_root, b2)
    acc3 = agg(h2, src, dst, w)
    return _tc_final(acc3, h2, W3_rel, W3_root, b3, batch,
                     Wm1, bm1, Wm2, bm2, Wm3, bm3)


# speedup vs baseline: 1.2942x; 1.0462x over previous
import functools

import jax
import jax.numpy as jnp
from jax import lax
from jax.experimental import pallas as pl
from jax.experimental.pallas import tpu as pltpu
from jax.experimental.pallas import tpu_sc as plsc

_N = 10000
_E = 320000
_G = 64
_NC = 2
_NS = 16
_NW = _NC * _NS
_CH = 64
_N0 = 245
_N1 = 70
_MAXCH = max(_N0, _N1)
_EPAD = _NS * (_N0 + _N1) * _CH
_NBUF = 5
_DIST = 4
_SR = 624
_TAIL = _N - _SR * _NS


def _make_edge_agg(h):
    mesh = plsc.VectorSubcoreMesh(core_axis_name="c", subcore_axis_name="s")

    @functools.partial(
        pl.kernel,
        out_type=jax.ShapeDtypeStruct((_NC, _N, h), jnp.float32),
        mesh=mesh,
        scratch_types=[
            pltpu.VMEM((_NBUF, _CH), jnp.int32),
            pltpu.VMEM((_NBUF, _CH), jnp.int32),
            pltpu.VMEM((_NBUF, _CH), jnp.float32),
            pltpu.VMEM((_CH, h), jnp.float32),
            pltpu.VMEM((_CH, h), jnp.float32),
            pltpu.VMEM((_CH, h), jnp.float32),
            pltpu.VMEM((_CH, h), jnp.float32),
            pltpu.VMEM((_CH, h), jnp.float32),
            pltpu.VMEM_SHARED((_N, h), jnp.float32),
            pltpu.SemaphoreType.DMA,
            pltpu.SemaphoreType.DMA,
            pltpu.SemaphoreType.DMA,
            pltpu.SemaphoreType.DMA,
            pltpu.SemaphoreType.DMA,
            pltpu.SemaphoreType.DMA,
            pltpu.SemaphoreType.DMA,
            pltpu.SemaphoreType.DMA,
            pltpu.SemaphoreType.DMA,
            pltpu.SemaphoreType.DMA,
            pltpu.SemaphoreType.DMA,
            pltpu.SemaphoreType.DMA,
            pltpu.SemaphoreType.DMA,
            pltpu.SemaphoreType.DMA,
            pltpu.SemaphoreType.DMA,
            pltpu.SemaphoreType.DMA,
            pltpu.SemaphoreType.DMA,
            pltpu.SemaphoreType.DMA,
            pltpu.SemaphoreType.DMA,
            pltpu.SemaphoreType.DMA,
        ],
    )
    def edge_agg(y_hbm, src_hbm, dst_hbm, w_hbm, out_hbm,
                 srcb_v, dstb_v, wb_v, r0, r1, r2, r3, r4, acc_sh,
                 sg0, sg1, sg2, sg3, sg4, ss0, ss1, ss2, ss3, ss4,
                 sc0, sc1, sc2, sc3, sc4, sd0, sd1, sd2, sd3, sd4):
        rows = (r0, r1, r2, r3, r4)
        sg = (sg0, sg1, sg2, sg3, sg4)
        ss = (ss0, ss1, ss2, ss3, ss4)
        sc = (sc0, sc1, sc2, sc3, sc4)
        sd = (sd0, sd1, sd2, sd3, sd4)
        cid = lax.axis_index("c")
        sid = lax.axis_index("s")
        zero16 = jnp.zeros((16,), jnp.float32)

        nch = jnp.where(cid == 0, _N0, _N1)
        cbase = cid * _NS * _N0 + sid * nch

        def eoff(j):
            return pl.multiple_of((cbase + j) * _CH, 8)

        def srcload(j, b):
            return pltpu.make_async_copy(src_hbm.at[pl.ds(eoff(j), _CH)],
                                         srcb_v.at[b], sc[b])

        def dload(j, b):
            return pltpu.make_async_copy(dst_hbm.at[pl.ds(eoff(j), _CH)],
                                         dstb_v.at[b], sd[b])

        def wload(j, b):
            return pltpu.make_async_copy(w_hbm.at[pl.ds(eoff(j), _CH)],
                                         wb_v.at[b], sd[b])

        for b in range(_NBUF):
            srcload(b, b).start()
        for b in range(_DIST):
            dload(b, b).start()
            wload(b, b).start()

        def zrow(rr, carry):
            for kk in range(h // 16):
                r0[rr, pl.ds(kk * 16, 16)] = zero16
            return carry
        lax.fori_loop(0, _CH, zrow, 0)

        row0 = sid * _SR
        nfull = _SR // _CH
        rem = _SR - nfull * _CH
        for i in range(nfull):
            pltpu.make_async_copy(
                r0, acc_sh.at[pl.ds(row0 + i * _CH, _CH)], ss0).start()
        pltpu.make_async_copy(
            r0.at[pl.ds(0, rem)],
            acc_sh.at[pl.ds(row0 + nfull * _CH, rem)], ss0).start()
        for i in range(nfull):
            pltpu.make_async_copy(
                r0, acc_sh.at[pl.ds(row0 + i * _CH, _CH)], ss0).wait()
        pltpu.make_async_copy(
            r0.at[pl.ds(0, rem)],
            acc_sh.at[pl.ds(row0 + nfull * _CH, rem)], ss0).wait()

        @pl.when(sid == 0)
        def _zero_tail():
            pltpu.sync_copy(r0.at[pl.ds(0, _TAIL)],
                            acc_sh.at[pl.ds(_SR * _NS, _TAIL)])
        plsc.subcore_barrier()

        def gat(j, b):
            return pltpu.make_async_copy(y_hbm.at[srcb_v.at[b]], rows[b],
                                         sg[b])

        def scat(b):
            return pltpu.make_async_copy(rows[b], acc_sh.at[dstb_v.at[b]],
                                         ss[b])

        def scale(b):
            rb = rows[b]

            def grp(g, c2):
                wv = wb_v[b, pl.ds(g * 16, 16)]
                for lane in range(16):
                    wgt = wv[lane]
                    e = g * 16 + lane
                    for kk in range(h // 16):
                        csl = pl.ds(kk * 16, 16)
                        rb[e, csl] = rb[e, csl] * wgt
                return c2
            lax.fori_loop(0, _CH // 16, grp, 0)

        for b in range(_DIST):
            srcload(b, b).wait()
            gat(b, b).start()

        def body(jj, carry):
            for b in range(_NBUF):
                j = jj * _NBUF + b

                @pl.when(j < nch)
                def _step():
                    gat(j, b).wait()
                    dload(j, b).wait()
                    wload(j, b).wait()
                    scale(b)
                    scat(b).start(add=True)
                    jf = j + _DIST
                    bf = (b + _DIST) % _NBUF

                    @pl.when(jf < nch)
                    def _issue():
                        @pl.when(j >= 1)
                        def _drain():
                            scat(bf).wait()
                        srcload(jf, bf).wait()
                        gat(jf, bf).start()
                        dload(jf, bf).start()
                        wload(jf, bf).start()

                    @pl.when(j + _NBUF < nch)
                    def _prefetch_src():
                        srcload(j + _NBUF, b).start()
            return carry
        lax.fori_loop(0, _MAXCH // _NBUF, body, 0)

        for b in range(_NBUF):
            scat(b).wait()
        plsc.subcore_barrier()

        for i in range(nfull + 1):
            cnt = _CH if i < nfull else rem
            b = i % 2
            row = row0 + i * _CH
            if i >= 2:
                pcnt = _CH if i - 2 < nfull else rem
                prow = row0 + (i - 2) * _CH
                pltpu.make_async_copy(
                    rows[b].at[pl.ds(0, pcnt)],
                    out_hbm.at[cid, pl.ds(prow, pcnt)], sg[b]).wait()
            pltpu.sync_copy(acc_sh.at[pl.ds(row, cnt)],
                            rows[b].at[pl.ds(0, cnt)])
            pltpu.make_async_copy(rows[b].at[pl.ds(0, cnt)],
                                  out_hbm.at[cid, pl.ds(row, cnt)],
                                  sg[b]).start()
        for i in (nfull - 1, nfull):
            cnt = _CH if i < nfull else rem
            b = i % 2
            row = row0 + i * _CH
            pltpu.make_async_copy(rows[b].at[pl.ds(0, cnt)],
                                  out_hbm.at[cid, pl.ds(row, cnt)],
                                  sg[b]).wait()

        @pl.when(sid == 0)
        def _read_tail():
            pltpu.sync_copy(acc_sh.at[pl.ds(_SR * _NS, _TAIL)],
                            r2.at[pl.ds(0, _TAIL)])
            pltpu.sync_copy(r2.at[pl.ds(0, _TAIL)],
                            out_hbm.at[cid, pl.ds(_SR * _NS, _TAIL)])

    return edge_agg


_edge_agg_cache = {}


def _edge_agg(h):
    if h not in _edge_agg_cache:
        _edge_agg_cache[h] = _make_edge_agg(h)
    return _edge_agg_cache[h]


def _tc_layer(acc, hprev, w_rel, w_root, b):
    hy = w_rel.shape[1]

    def body(a_ref, h_ref, wr_ref, wo_ref, b_ref, o_ref):
        agg = a_ref[0] + a_ref[1]
        o_ref[...] = jnp.tanh(
            jnp.dot(agg, wr_ref[...], preferred_element_type=jnp.float32)
            + b_ref[...]
            + jnp.dot(h_ref[...], wo_ref[...],
                      preferred_element_type=jnp.float32))

    return pl.pallas_call(
        body,
        out_shape=jax.ShapeDtypeStruct((_N, hy), jnp.float32),
    )(acc, hprev, w_rel, w_root, b.reshape(1, hy))


def _tc_final(acc, hprev, w_rel, w_root, b, batch, wm1, bm1, wm2, bm2,
              wm3, bm3):
    def body(a_ref, h_ref, wr_ref, wo_ref, b_ref, batch_ref,
             w1_ref, b1_ref, w2_ref, b2_ref, w3_ref, b3_ref, out_ref):
        agg = a_ref[0] + a_ref[1]
        hh = jnp.tanh(
            jnp.dot(agg, wr_ref[...], preferred_element_type=jnp.float32)
            + b_ref[...]
            + jnp.dot(h_ref[...], wo_ref[...],
                      preferred_element_type=jnp.float32))
        gids = lax.broadcasted_iota(jnp.int32, (_G, _N), 0)
        onehot = (batch_ref[...] == gids).astype(jnp.float32)
        pooled = jnp.dot(onehot, hh, preferred_element_type=jnp.float32)
        z1 = jnp.maximum(
            jnp.dot(pooled, w1_ref[...], preferred_element_type=jnp.float32)
            + b1_ref[...], 0.0)
        z2 = jnp.maximum(
            jnp.dot(z1, w2_ref[...], preferred_element_type=jnp.float32)
            + b2_ref[...], 0.0)
        out_ref[...] = (jnp.dot(z2, w3_ref[...], preferred_element_type=jnp.float32)
                        + b3_ref[...])

    return pl.pallas_call(
        body,
        out_shape=jax.ShapeDtypeStruct((_G, 1), jnp.float32),
    )(acc, hprev, w_rel, w_root, b.reshape(1, -1), batch.reshape(1, _N),
      wm1, bm1.reshape(1, -1), wm2, bm2.reshape(1, -1), wm3,
      bm3.reshape(1, 1))


def kernel(x, edge_index, batch, edge_attr,
           W1_rel, b1, W1_root, W2_rel, b2, W2_root, W3_rel, b3, W3_root,
           Wm1, bm1, Wm2, bm2, Wm3, bm3):
    pad = _EPAD - _E
    src = jnp.concatenate([edge_index[0], jnp.zeros((pad,), jnp.int32)])
    dst = jnp.concatenate([edge_index[1], jnp.zeros((pad,), jnp.int32)])
    w = jnp.concatenate([edge_attr, jnp.zeros((pad,), jnp.float32)])

    agg = _edge_agg(128)
    acc1 = agg(x, src, dst, w)
    h1 = _tc_layer(acc1, x, W1_rel, W1_root, b1)
    acc2 = agg(h1, src, dst, w)
    h2 = _tc_layer(acc2, h1, W2_rel, W2_root, b2)
    acc3 = agg(h2, src, dst, w)
    return _tc_final(acc3, h2, W3_rel, W3_root, b3, batch,
                     Wm1, bm1, Wm2, bm2, Wm3, bm3)
